# baseline (speedup 1.0000x reference)
import jax
import jax.numpy as jnp
from jax.experimental import pallas as pl

EPS_BN = 1e-5


def _lin(p, x):
    return x @ p["W"] + p["b"]


def _bn(x, p):
    mean = jnp.mean(x, axis=0)
    var = jnp.var(x, axis=0)
    return p["gamma"] * (x - mean) / jnp.sqrt(var + EPS_BN) + p["beta"]


def _layer(lp, h, e, src, dst, n_nodes):
    h_in, e_in = h, e
    Ah = _lin(lp["A"], h)
    Bh = _lin(lp["B"], h)
    Dh = _lin(lp["D"], h)
    Eh = _lin(lp["E"], h)
    Ce = _lin(lp["C"], e)
    e_new = Ce + Dh[src] + Eh[dst]
    sigma = jax.nn.sigmoid(e_new)
    num = jax.ops.segment_sum(sigma * Bh[src], dst, num_segments=n_nodes)
    den = jax.ops.segment_sum(sigma, dst, num_segments=n_nodes)
    h_new = Ah + num / (den + 1e-6)
    h_new = jax.nn.relu(_bn(h_new, lp["bn_h"]))
    e_out = jax.nn.relu(_bn(e_new, lp["bn_e"]))
    return h_in + h_new, e_in + e_out


def _mlp_head_kernel(hg_ref, w0_ref, b0_ref, w1_ref, b1_ref, w2_ref, b2_ref,
                     o_ref):
    y = hg_ref[...]
    y = jax.nn.relu(y @ w0_ref[...] + b0_ref[...])
    y = jax.nn.relu(y @ w1_ref[...] + b1_ref[...])
    o_ref[...] = y @ w2_ref[...] + b2_ref[...]


def _mlp_head(hg, mlp):
    n_classes = mlp[2]["b"].shape[0]
    return pl.pallas_call(
        _mlp_head_kernel,
        out_shape=jax.ShapeDtypeStruct((1, n_classes), jnp.float32),
    )(hg, mlp[0]["W"], mlp[0]["b"].reshape(1, -1),
      mlp[1]["W"], mlp[1]["b"].reshape(1, -1),
      mlp[2]["W"], mlp[2]["b"].reshape(1, -1))


def kernel(h, e, edge_index, params):
    src = edge_index[0]
    dst = edge_index[1]
    n_nodes = h.shape[0]
    h = _lin(params["emb_h"], h)
    e = _lin(params["emb_e"], e)
    for lp in params["layers"]:
        h, e = _layer(lp, h, e, src, dst, n_nodes)
    hg = jnp.mean(h, axis=0, keepdims=True)
    return _mlp_head(hg, params["mlp"])


# speedup vs baseline: 2.2402x; 2.2402x over previous
import jax
import jax.numpy as jnp
from jax import lax
from jax.experimental import pallas as pl
from jax.experimental.pallas import tpu as pltpu
from jax.experimental.pallas import tpu_sc as plsc

N_NODES = 10000
N_EDGES = 320000
PE = N_EDGES // 2
DIM = 128
HALF = 64
EPS_BN = 1e-5
NS = 16
L = 16
K = 128
NCH = PE // K
ROWS_PT = 632
N_ACC = NS * ROWS_PT

EDGE_BLK = 2000
EG = N_EDGES // EDGE_BLK
PG = PE // EDGE_BLK
NODE_BLK = 2000
NG = N_NODES // NODE_BLK

_F32 = jnp.float32


def _mm_kernel(x_ref, w_ref, b_ref, o_ref):
    o_ref[...] = (
        jnp.dot(x_ref[...], w_ref[...], preferred_element_type=_F32)
        + b_ref[...]
    )


def _edge_emb(e, p):
    return pl.pallas_call(
        _mm_kernel,
        grid=(EG,),
        in_specs=[
            pl.BlockSpec((EDGE_BLK, DIM), lambda i: (i, 0)),
            pl.BlockSpec((DIM, DIM), lambda i: (0, 0)),
            pl.BlockSpec((1, DIM), lambda i: (0, 0)),
        ],
        out_specs=pl.BlockSpec((EDGE_BLK, DIM), lambda i: (i, 0)),
        out_shape=jax.ShapeDtypeStruct((N_EDGES, DIM), _F32),
    )(e, p["W"], p["b"].reshape(1, DIM))


def _node_emb(h, p):
    return pl.pallas_call(
        _mm_kernel,
        out_shape=jax.ShapeDtypeStruct((N_NODES, DIM), _F32),
    )(h, p["W"], p["b"].reshape(1, DIM))


def _split_w(w):
    return jnp.stack([w[:, :HALF], w[:, HALF:]])


def _ce_kernel(x1_ref, x2_ref, w_ref, b_ref, o_ref):
    x1 = jnp.dot(x1_ref[...], w_ref[0], preferred_element_type=_F32)
    x2 = jnp.dot(x2_ref[...], w_ref[0], preferred_element_type=_F32)
    o_ref[0] = jnp.concatenate([x1, x2], axis=1) + b_ref[0]


def _ce_packed(e, w, b):
    b_half = b.reshape(2, HALF)
    b2 = jnp.concatenate([b_half, b_half], axis=1).reshape(2, 1, DIM)
    return pl.pallas_call(
        _ce_kernel,
        grid=(PG, 2),
        in_specs=[
            pl.BlockSpec((EDGE_BLK, DIM), lambda i, c: (i, 0)),
            pl.BlockSpec((EDGE_BLK, DIM), lambda i, c: (PG + i, 0)),
            pl.BlockSpec((1, DIM, HALF), lambda i, c: (c, 0, 0)),
            pl.BlockSpec((1, 1, DIM), lambda i, c: (c, 0, 0)),
        ],
        out_specs=pl.BlockSpec((1, EDGE_BLK, DIM), lambda i, c: (c, i, 0)),
        out_shape=jax.ShapeDtypeStruct((2, PE, DIM), _F32),
    )(e, e, _split_w(w), b2)


def _node_prep_kernel(h_ref, wa_ref, ba_ref, wb_ref, wd_ref, we_ref,
                      ah_ref, db_ref, eh_ref):
    h = h_ref[...]
    ah_ref[...] = (
        jnp.dot(h, wa_ref[...], preferred_element_type=_F32) + ba_ref[...]
    )
    bh = jnp.dot(h, wb_ref[0], preferred_element_type=_F32)
    dh = jnp.dot(h, wd_ref[0], preferred_element_type=_F32)
    ehf = jnp.dot(h, we_ref[...], preferred_element_type=_F32)
    db_ref[0] = jnp.concatenate([dh, bh], axis=1)
    c = pl.program_id(1)
    eh_roll = jnp.concatenate([ehf[:, HALF:], ehf[:, :HALF]], axis=1)
    eh_ref[0] = jnp.where(c == 0, ehf, eh_roll)


def _node_prep(h, lp):
    return pl.pallas_call(
        _node_prep_kernel,
        grid=(NG, 2),
        in_specs=[
            pl.BlockSpec((NODE_BLK, DIM), lambda i, c: (i, 0)),
            pl.BlockSpec((DIM, DIM), lambda i, c: (0, 0)),
            pl.BlockSpec((1, DIM), lambda i, c: (0, 0)),
            pl.BlockSpec((1, DIM, HALF), lambda i, c: (c, 0, 0)),
            pl.BlockSpec((1, DIM, HALF), lambda i, c: (c, 0, 0)),
            pl.BlockSpec((DIM, DIM), lambda i, c: (0, 0)),
        ],
        out_specs=[
            pl.BlockSpec((NODE_BLK, DIM), lambda i, c: (i, 0)),
            pl.BlockSpec((1, NODE_BLK, DIM), lambda i, c: (c, i, 0)),
            pl.BlockSpec((1, NODE_BLK, DIM), lambda i, c: (c, i, 0)),
        ],
        out_shape=[
            jax.ShapeDtypeStruct((N_NODES, DIM), _F32),
            jax.ShapeDtypeStruct((2, N_NODES, DIM), _F32),
            jax.ShapeDtypeStruct((2, N_NODES, DIM), _F32),
        ],
    )(h, lp["A"]["W"], lp["A"]["b"].reshape(1, DIM),
      _split_w(lp["B"]["W"]), _split_w(lp["D"]["W"]), lp["E"]["W"])


def _node_update_core(ah_ref, nd_ref, hin_ref, g_ref, b_ref, bb_ref):
    nd = nd_ref[...]
    den = jnp.concatenate(
        [nd[0, :N_NODES, :HALF], nd[1, :N_NODES, :HALF]], axis=1)
    num = jnp.concatenate(
        [nd[0, :N_NODES, HALF:], nd[1, :N_NODES, HALF:]], axis=1)
    x = ah_ref[...] + (num + bb_ref[...] * den) / (den + 1e-6)
    mean = jnp.mean(x, axis=0, keepdims=True)
    var = jnp.mean(x * x, axis=0, keepdims=True) - mean * mean
    xn = g_ref[...] * (x - mean) * lax.rsqrt(var + EPS_BN) + b_ref[...]
    return hin_ref[...] + jnp.maximum(xn, 0.0)


def _node_update_kernel(ah_ref, nd_ref, hin_ref, g_ref, b_ref, bb_ref,
                        o_ref):
    o_ref[...] = _node_update_core(ah_ref, nd_ref, hin_ref, g_ref, b_ref,
                                   bb_ref)


def _node_update(ah, nd, h_in, bn, b_B):
    return pl.pallas_call(
        _node_update_kernel,
        out_shape=jax.ShapeDtypeStruct((N_NODES, DIM), _F32),
    )(ah, nd, h_in, bn["gamma"].reshape(1, DIM), bn["beta"].reshape(1, DIM),
      b_B.reshape(1, DIM))


def _node_final_kernel(ah_ref, nd_ref, hin_ref, g_ref, b_ref, bb_ref,
                       w0_ref, b0_ref, w1_ref, b1_ref, w2_ref, b2_ref,
                       o_ref):
    hout = _node_update_core(ah_ref, nd_ref, hin_ref, g_ref, b_ref, bb_ref)
    hg = jnp.mean(hout, axis=0, keepdims=True)
    y = jnp.maximum(
        jnp.dot(hg, w0_ref[...], preferred_element_type=_F32) + b0_ref[...],
        0.0)
    y = jnp.maximum(
        jnp.dot(y, w1_ref[...], preferred_element_type=_F32) + b1_ref[...],
        0.0)
    o_ref[...] = (
        jnp.dot(y, w2_ref[...], preferred_element_type=_F32) + b2_ref[...]
    )


def _node_final(ah, nd, h_in, bn, b_B, mlp):
    n_classes = mlp[2]["b"].shape[0]
    return pl.pallas_call(
        _node_final_kernel,
        out_shape=jax.ShapeDtypeStruct((1, n_classes), _F32),
    )(ah, nd, h_in, bn["gamma"].reshape(1, DIM), bn["beta"].reshape(1, DIM),
      b_B.reshape(1, DIM),
      mlp[0]["W"], mlp[0]["b"].reshape(1, -1),
      mlp[1]["W"], mlp[1]["b"].reshape(1, -1),
      mlp[2]["W"], mlp[2]["b"].reshape(1, -1))


def _e_stats_kernel(e0_ref, e1_ref, o_ref):
    i = pl.program_id(0)
    b0 = e0_ref[0]
    b1 = e1_ref[0]
    s0 = jnp.sum(b0[:, :HALF], axis=0) + jnp.sum(b0[:, HALF:], axis=0)
    s1 = jnp.sum(b1[:, :HALF], axis=0) + jnp.sum(b1[:, HALF:], axis=0)
    q0 = (jnp.sum(b0[:, :HALF] * b0[:, :HALF], axis=0)
          + jnp.sum(b0[:, HALF:] * b0[:, HALF:], axis=0))
    q1 = (jnp.sum(b1[:, :HALF] * b1[:, :HALF], axis=0)
          + jnp.sum(b1[:, HALF:] * b1[:, HALF:], axis=0))
    blk = jnp.stack([jnp.concatenate([s0, s1]), jnp.concatenate([q0, q1])])

    @pl.when(i == 0)
    def _():
        o_ref[...] = blk

    @pl.when(i > 0)
    def _():
        o_ref[...] += blk


def _e_stats(enew):
    return pl.pallas_call(
        _e_stats_kernel,
        grid=(PG,),
        in_specs=[
            pl.BlockSpec((1, EDGE_BLK, DIM), lambda i: (0, i, 0)),
            pl.BlockSpec((1, EDGE_BLK, DIM), lambda i: (1, i, 0)),
        ],
        out_specs=pl.BlockSpec((2, DIM), lambda i: (0, 0)),
        out_shape=jax.ShapeDtypeStruct((2, DIM), _F32),
    )(enew, enew)


def _e_update_kernel(ein_ref, e0_ref, e1_ref, st_ref, g_ref, b_ref, o_ref):
    front = pl.program_id(0) < PG
    b0 = e0_ref[0]
    b1 = e1_ref[0]
    en_c0 = jnp.where(front, b0[:, :HALF], b0[:, HALF:])
    en_c1 = jnp.where(front, b1[:, :HALF], b1[:, HALF:])
    en = jnp.concatenate([en_c0, en_c1], axis=1)
    st = st_ref[...]
    mean = st[0:1] / N_EDGES
    var = st[1:2] / N_EDGES - mean * mean
    xn = g_ref[...] * (en - mean) * lax.rsqrt(var + EPS_BN) + b_ref[...]
    o_ref[...] = ein_ref[...] + jnp.maximum(xn, 0.0)


def _e_update(e_in, enew, st, bn):
    return pl.pallas_call(
        _e_update_kernel,
        grid=(EG,),
        in_specs=[
            pl.BlockSpec((EDGE_BLK, DIM), lambda i: (i, 0)),
            pl.BlockSpec((1, EDGE_BLK, DIM), lambda i: (0, i % PG, 0)),
            pl.BlockSpec((1, EDGE_BLK, DIM), lambda i: (1, i % PG, 0)),
            pl.BlockSpec((2, DIM), lambda i: (0, 0)),
            pl.BlockSpec((1, DIM), lambda i: (0, 0)),
            pl.BlockSpec((1, DIM), lambda i: (0, 0)),
        ],
        out_specs=pl.BlockSpec((EDGE_BLK, DIM), lambda i: (i, 0)),
        out_shape=jax.ShapeDtypeStruct((N_EDGES, DIM), _F32),
    )(e_in, enew, enew, st, bn["gamma"].reshape(1, DIM),
      bn["beta"].reshape(1, DIM))


def _sc_message(ce, src, dst, db, eh, write_enew):
    mesh = plsc.VectorSubcoreMesh(core_axis_name="c", subcore_axis_name="s")
    out_type = []
    if write_enew:
        out_type.append(jax.ShapeDtypeStruct((2 * PE, DIM), _F32))
    out_type.append(jax.ShapeDtypeStruct((2 * N_ACC, DIM), _F32))
    scratch_types = [
        pltpu.VMEM((K,), jnp.int32),
        pltpu.VMEM((K,), jnp.int32),
        pltpu.VMEM((K,), jnp.int32),
        pltpu.VMEM((K, DIM), _F32),
        pltpu.VMEM((K, DIM), _F32),
        pltpu.VMEM((K, DIM), _F32),
        pltpu.VMEM_SHARED((N_ACC, DIM), _F32),
    ]

    def body(ce_hbm, src_hbm, dst_hbm, db_hbm, eh_hbm, *refs):
        if write_enew:
            enew_hbm, nd_hbm = refs[0], refs[1]
            scr = refs[2:]
        else:
            enew_hbm, nd_hbm = None, refs[0]
            scr = refs[1:]
        (src_v, dst_v, gdst_v, ce_v, db_v, ed_v, acc) = scr
        c = lax.axis_index("c")
        s = lax.axis_index("s")

        @pl.loop(0, K)
        def _(i):
            for j in range(DIM // L):
                ed_v[i, pl.ds(j * L, L)] = jnp.zeros((L,), _F32)

        for t in range(ROWS_PT // K):
            pltpu.sync_copy(ed_v, acc.at[pl.ds(s * ROWS_PT + t * K, K)])
        _rem = ROWS_PT % K
        if _rem:
            pltpu.sync_copy(
                ed_v.at[pl.ds(0, _rem)],
                acc.at[pl.ds(s * ROWS_PT + (ROWS_PT // K) * K, _rem)])
        plsc.subcore_barrier()

        coff = c * N_NODES
        eoff = c * PE
        g0 = (NCH * s) // NS
        g1 = (NCH * (s + 1)) // NS

        def chunk(g, carry):
            base = g * K
            pltpu.sync_copy(ce_hbm.at[pl.ds(eoff + base, K)], ce_v)
            for p in range(2):
                pltpu.sync_copy(src_hbm.at[pl.ds(p * PE + base, K)], src_v)
                pltpu.sync_copy(dst_hbm.at[pl.ds(p * PE + base, K)], dst_v)
                for j in range(K // L):
                    sl = pl.ds(j * L, L)
                    src_v[sl] = src_v[sl] + coff
                    gdst_v[sl] = dst_v[sl] + coff
                pltpu.sync_copy(db_hbm.at[src_v], db_v)
                pltpu.sync_copy(eh_hbm.at[gdst_v], ed_v)

                def row(i, rc):
                    for j in range(HALF // L):
                        slp = pl.ds(p * HALF + j * L, L)
                        sl = pl.ds(j * L, L)
                        sl2 = pl.ds(HALF + j * L, L)
                        en = ce_v[i, slp] + db_v[i, sl] + ed_v[i, sl]
                        if write_enew:
                            ce_v[i, slp] = en
                        sg = 1.0 / (1.0 + jnp.exp(-en))
                        ed_v[i, sl] = sg
                        ed_v[i, sl2] = sg * db_v[i, sl2]
                    return rc

                lax.fori_loop(0, K, row, 0)
                pltpu.sync_copy(ed_v, acc.at[dst_v], add=True)
            if write_enew:
                pltpu.sync_copy(ce_v, enew_hbm.at[pl.ds(eoff + base, K)])
            return carry

        lax.fori_loop(g0, g1, chunk, 0)
        plsc.subcore_barrier()
        pltpu.sync_copy(
            acc.at[pl.ds(s * ROWS_PT, ROWS_PT)],
            nd_hbm.at[pl.ds(c * N_ACC + s * ROWS_PT, ROWS_PT)])

    fn = pl.kernel(body, out_type=out_type, mesh=mesh,
                   scratch_types=scratch_types)
    res = fn(ce, src, dst, db, eh)
    return tuple(res) if write_enew else res[0]


def kernel(h, e, edge_index, params):
    src = edge_index[0].astype(jnp.int32)
    dst = edge_index[1].astype(jnp.int32)
    h = _node_emb(h, params["emb_h"])
    e = _edge_emb(e, params["emb_e"])
    layers = params["layers"]
    for l, lp in enumerate(layers):
        last = l == len(layers) - 1
        ah, db, eh = _node_prep(h, lp)
        ce = _ce_packed(
            e, lp["C"]["W"], lp["C"]["b"] + lp["D"]["b"] + lp["E"]["b"],
        ).reshape(2 * PE, DIM)
        db = db.reshape(2 * N_NODES, DIM)
        eh = eh.reshape(2 * N_NODES, DIM)
        if last:
            nd = _sc_message(ce, src, dst, db, eh, write_enew=False)
            return _node_final(ah, nd.reshape(2, N_ACC, DIM), h,
                               lp["bn_h"], lp["B"]["b"], params["mlp"])
        enew, nd = _sc_message(ce, src, dst, db, eh, write_enew=True)
        h = _node_update(ah, nd.reshape(2, N_ACC, DIM), h, lp["bn_h"],
                         lp["B"]["b"])
        enew = enew.reshape(2, PE, DIM)
        st = _e_stats(enew)
        e = _e_update(e, enew, st, lp["bn_e"])
    return None


# speedup vs baseline: 2.7114x; 1.2104x over previous
import jax
import jax.numpy as jnp
from jax import lax
from jax.experimental import pallas as pl
from jax.experimental.pallas import tpu as pltpu
from jax.experimental.pallas import tpu_sc as plsc

N_NODES = 10000
N_EDGES = 320000
PE = N_EDGES // 2
DIM = 128
HALF = 64
EPS_BN = 1e-5
NS = 16
L = 16
K = 64
NCH = PE // K
ROWS_PT = 632
N_ACC = NS * ROWS_PT

EDGE_BLK = 2000
EG = N_EDGES // EDGE_BLK
PG = PE // EDGE_BLK
NODE_BLK = 2000
NG = N_NODES // NODE_BLK

_F32 = jnp.float32


def _mm_kernel(x_ref, w_ref, b_ref, o_ref):
    o_ref[...] = (
        jnp.dot(x_ref[...], w_ref[...], preferred_element_type=_F32)
        + b_ref[...]
    )


def _edge_emb(e, p):
    return pl.pallas_call(
        _mm_kernel,
        grid=(EG,),
        in_specs=[
            pl.BlockSpec((EDGE_BLK, DIM), lambda i: (i, 0)),
            pl.BlockSpec((DIM, DIM), lambda i: (0, 0)),
            pl.BlockSpec((1, DIM), lambda i: (0, 0)),
        ],
        out_specs=pl.BlockSpec((EDGE_BLK, DIM), lambda i: (i, 0)),
        out_shape=jax.ShapeDtypeStruct((N_EDGES, DIM), _F32),
    )(e, p["W"], p["b"].reshape(1, DIM))


def _node_emb(h, p):
    return pl.pallas_call(
        _mm_kernel,
        out_shape=jax.ShapeDtypeStruct((N_NODES, DIM), _F32),
    )(h, p["W"], p["b"].reshape(1, DIM))


def _split_w(w):
    return jnp.stack([w[:, :HALF], w[:, HALF:]])


def _ce_kernel(x1_ref, x2_ref, w_ref, b_ref, o_ref):
    x1 = jnp.dot(x1_ref[...], w_ref[0], preferred_element_type=_F32)
    x2 = jnp.dot(x2_ref[...], w_ref[0], preferred_element_type=_F32)
    o_ref[0] = jnp.concatenate([x1, x2], axis=1) + b_ref[0]


def _ce_packed(e, w, b):
    b_half = b.reshape(2, HALF)
    b2 = jnp.concatenate([b_half, b_half], axis=1).reshape(2, 1, DIM)
    return pl.pallas_call(
        _ce_kernel,
        grid=(PG, 2),
        in_specs=[
            pl.BlockSpec((EDGE_BLK, DIM), lambda i, c: (i, 0)),
            pl.BlockSpec((EDGE_BLK, DIM), lambda i, c: (PG + i, 0)),
            pl.BlockSpec((1, DIM, HALF), lambda i, c: (c, 0, 0)),
            pl.BlockSpec((1, 1, DIM), lambda i, c: (c, 0, 0)),
        ],
        out_specs=pl.BlockSpec((1, EDGE_BLK, DIM), lambda i, c: (c, i, 0)),
        out_shape=jax.ShapeDtypeStruct((2, PE, DIM), _F32),
    )(e, e, _split_w(w), b2)


def _node_prep_kernel(h_ref, wa_ref, ba_ref, wb_ref, wd_ref, we_ref,
                      ah_ref, db_ref, eh_ref):
    h = h_ref[...]
    ah_ref[...] = (
        jnp.dot(h, wa_ref[...], preferred_element_type=_F32) + ba_ref[...]
    )
    bh = jnp.dot(h, wb_ref[0], preferred_element_type=_F32)
    dh = jnp.dot(h, wd_ref[0], preferred_element_type=_F32)
    ehf = jnp.dot(h, we_ref[...], preferred_element_type=_F32)
    db_ref[0] = jnp.concatenate([dh, bh], axis=1)
    c = pl.program_id(1)
    eh_roll = jnp.concatenate([ehf[:, HALF:], ehf[:, :HALF]], axis=1)
    eh_ref[0] = jnp.where(c == 0, ehf, eh_roll)


def _node_prep(h, lp):
    return pl.pallas_call(
        _node_prep_kernel,
        grid=(NG, 2),
        in_specs=[
            pl.BlockSpec((NODE_BLK, DIM), lambda i, c: (i, 0)),
            pl.BlockSpec((DIM, DIM), lambda i, c: (0, 0)),
            pl.BlockSpec((1, DIM), lambda i, c: (0, 0)),
            pl.BlockSpec((1, DIM, HALF), lambda i, c: (c, 0, 0)),
            pl.BlockSpec((1, DIM, HALF), lambda i, c: (c, 0, 0)),
            pl.BlockSpec((DIM, DIM), lambda i, c: (0, 0)),
        ],
        out_specs=[
            pl.BlockSpec((NODE_BLK, DIM), lambda i, c: (i, 0)),
            pl.BlockSpec((1, NODE_BLK, DIM), lambda i, c: (c, i, 0)),
            pl.BlockSpec((1, NODE_BLK, DIM), lambda i, c: (c, i, 0)),
        ],
        out_shape=[
            jax.ShapeDtypeStruct((N_NODES, DIM), _F32),
            jax.ShapeDtypeStruct((2, N_NODES, DIM), _F32),
            jax.ShapeDtypeStruct((2, N_NODES, DIM), _F32),
        ],
    )(h, lp["A"]["W"], lp["A"]["b"].reshape(1, DIM),
      _split_w(lp["B"]["W"]), _split_w(lp["D"]["W"]), lp["E"]["W"])


def _node_update_core(ah_ref, nd_ref, hin_ref, g_ref, b_ref, bb_ref):
    nd = nd_ref[...]
    den = jnp.concatenate(
        [nd[0, :N_NODES, :HALF], nd[1, :N_NODES, :HALF]], axis=1)
    num = jnp.concatenate(
        [nd[0, :N_NODES, HALF:], nd[1, :N_NODES, HALF:]], axis=1)
    x = ah_ref[...] + (num + bb_ref[...] * den) / (den + 1e-6)
    mean = jnp.mean(x, axis=0, keepdims=True)
    var = jnp.mean(x * x, axis=0, keepdims=True) - mean * mean
    xn = g_ref[...] * (x - mean) * lax.rsqrt(var + EPS_BN) + b_ref[...]
    return hin_ref[...] + jnp.maximum(xn, 0.0)


def _node_update_kernel(ah_ref, nd_ref, hin_ref, g_ref, b_ref, bb_ref,
                        o_ref):
    o_ref[...] = _node_update_core(ah_ref, nd_ref, hin_ref, g_ref, b_ref,
                                   bb_ref)


def _node_update(ah, nd, h_in, bn, b_B):
    return pl.pallas_call(
        _node_update_kernel,
        out_shape=jax.ShapeDtypeStruct((N_NODES, DIM), _F32),
    )(ah, nd, h_in, bn["gamma"].reshape(1, DIM), bn["beta"].reshape(1, DIM),
      b_B.reshape(1, DIM))


def _node_final_kernel(ah_ref, nd_ref, hin_ref, g_ref, b_ref, bb_ref,
                       w0_ref, b0_ref, w1_ref, b1_ref, w2_ref, b2_ref,
                       o_ref):
    hout = _node_update_core(ah_ref, nd_ref, hin_ref, g_ref, b_ref, bb_ref)
    hg = jnp.mean(hout, axis=0, keepdims=True)
    y = jnp.maximum(
        jnp.dot(hg, w0_ref[...], preferred_element_type=_F32) + b0_ref[...],
        0.0)
    y = jnp.maximum(
        jnp.dot(y, w1_ref[...], preferred_element_type=_F32) + b1_ref[...],
        0.0)
    o_ref[...] = (
        jnp.dot(y, w2_ref[...], preferred_element_type=_F32) + b2_ref[...]
    )


def _node_final(ah, nd, h_in, bn, b_B, mlp):
    n_classes = mlp[2]["b"].shape[0]
    return pl.pallas_call(
        _node_final_kernel,
        out_shape=jax.ShapeDtypeStruct((1, n_classes), _F32),
    )(ah, nd, h_in, bn["gamma"].reshape(1, DIM), bn["beta"].reshape(1, DIM),
      b_B.reshape(1, DIM),
      mlp[0]["W"], mlp[0]["b"].reshape(1, -1),
      mlp[1]["W"], mlp[1]["b"].reshape(1, -1),
      mlp[2]["W"], mlp[2]["b"].reshape(1, -1))


def _e_stats_kernel(e0_ref, e1_ref, o_ref):
    i = pl.program_id(0)
    b0 = e0_ref[0]
    b1 = e1_ref[0]
    s0 = jnp.sum(b0[:, :HALF], axis=0) + jnp.sum(b0[:, HALF:], axis=0)
    s1 = jnp.sum(b1[:, :HALF], axis=0) + jnp.sum(b1[:, HALF:], axis=0)
    q0 = (jnp.sum(b0[:, :HALF] * b0[:, :HALF], axis=0)
          + jnp.sum(b0[:, HALF:] * b0[:, HALF:], axis=0))
    q1 = (jnp.sum(b1[:, :HALF] * b1[:, :HALF], axis=0)
          + jnp.sum(b1[:, HALF:] * b1[:, HALF:], axis=0))
    blk = jnp.stack([jnp.concatenate([s0, s1]), jnp.concatenate([q0, q1])])

    @pl.when(i == 0)
    def _():
        o_ref[...] = blk

    @pl.when(i > 0)
    def _():
        o_ref[...] += blk


def _e_stats(enew):
    return pl.pallas_call(
        _e_stats_kernel,
        grid=(PG,),
        in_specs=[
            pl.BlockSpec((1, EDGE_BLK, DIM), lambda i: (0, i, 0)),
            pl.BlockSpec((1, EDGE_BLK, DIM), lambda i: (1, i, 0)),
        ],
        out_specs=pl.BlockSpec((2, DIM), lambda i: (0, 0)),
        out_shape=jax.ShapeDtypeStruct((2, DIM), _F32),
    )(enew, enew)


def _e_update_kernel(ein_ref, e0_ref, e1_ref, st_ref, g_ref, b_ref, o_ref):
    front = pl.program_id(0) < PG
    b0 = e0_ref[0]
    b1 = e1_ref[0]
    en_c0 = jnp.where(front, b0[:, :HALF], b0[:, HALF:])
    en_c1 = jnp.where(front, b1[:, :HALF], b1[:, HALF:])
    en = jnp.concatenate([en_c0, en_c1], axis=1)
    st = st_ref[...]
    mean = st[0:1] / N_EDGES
    var = st[1:2] / N_EDGES - mean * mean
    xn = g_ref[...] * (en - mean) * lax.rsqrt(var + EPS_BN) + b_ref[...]
    o_ref[...] = ein_ref[...] + jnp.maximum(xn, 0.0)


def _e_update(e_in, enew, st, bn):
    return pl.pallas_call(
        _e_update_kernel,
        grid=(EG,),
        in_specs=[
            pl.BlockSpec((EDGE_BLK, DIM), lambda i: (i, 0)),
            pl.BlockSpec((1, EDGE_BLK, DIM), lambda i: (0, i % PG, 0)),
            pl.BlockSpec((1, EDGE_BLK, DIM), lambda i: (1, i % PG, 0)),
            pl.BlockSpec((2, DIM), lambda i: (0, 0)),
            pl.BlockSpec((1, DIM), lambda i: (0, 0)),
            pl.BlockSpec((1, DIM), lambda i: (0, 0)),
        ],
        out_specs=pl.BlockSpec((EDGE_BLK, DIM), lambda i: (i, 0)),
        out_shape=jax.ShapeDtypeStruct((N_EDGES, DIM), _F32),
    )(e_in, enew, enew, st, bn["gamma"].reshape(1, DIM),
      bn["beta"].reshape(1, DIM))


def _sc_message(ce, src, dst, db, eh, write_enew):
    mesh = plsc.VectorSubcoreMesh(core_axis_name="c", subcore_axis_name="s")
    out_type = []
    if write_enew:
        out_type.append(jax.ShapeDtypeStruct((2 * PE, DIM), _F32))
    out_type.append(jax.ShapeDtypeStruct((2 * N_ACC, DIM), _F32))
    scratch_types = [
        pltpu.VMEM((K,), jnp.int32),
        pltpu.VMEM((K,), jnp.int32),
        pltpu.VMEM((K,), jnp.int32),
        pltpu.VMEM((K,), jnp.int32),
        pltpu.VMEM((K,), jnp.int32),
        pltpu.VMEM((K,), jnp.int32),
        pltpu.VMEM((K, DIM), _F32),
        pltpu.VMEM((K, DIM), _F32),
        pltpu.VMEM((K, DIM), _F32),
        pltpu.VMEM((K, DIM), _F32),
        pltpu.VMEM((K, DIM), _F32),
        pltpu.VMEM_SHARED((N_ACC, DIM), _F32),
        pltpu.SemaphoreType.DMA,
        pltpu.SemaphoreType.DMA,
        pltpu.SemaphoreType.DMA,
        pltpu.SemaphoreType.DMA,
    ]

    def body(ce_hbm, src_hbm, dst_hbm, db_hbm, eh_hbm, *refs):
        if write_enew:
            enew_hbm, nd_hbm = refs[0], refs[1]
            scr = refs[2:]
        else:
            enew_hbm, nd_hbm = None, refs[0]
            scr = refs[1:]
        (src0, dst0, gd0, src1, dst1, gd1, ce_v, db0, ed0, db1, ed1,
         acc, sem_gf, sem_gb, sem_s, sem_i) = scr
        c = lax.axis_index("c")
        s = lax.axis_index("s")

        @pl.loop(0, K)
        def _(i):
            for j in range(DIM // L):
                ed0[i, pl.ds(j * L, L)] = jnp.zeros((L,), _F32)

        for t in range(ROWS_PT // K):
            pltpu.sync_copy(ed0, acc.at[pl.ds(s * ROWS_PT + t * K, K)])
        _rem = ROWS_PT % K
        if _rem:
            pltpu.sync_copy(
                ed0.at[pl.ds(0, _rem)],
                acc.at[pl.ds(s * ROWS_PT + (ROWS_PT // K) * K, _rem)])
        plsc.subcore_barrier()

        coff = c * N_NODES
        eoff = c * PE
        g0 = (NCH * s) // NS
        g1 = (NCH * (s + 1)) // NS

        idx_bufs = ((src0, dst0), (src1, dst1))

        def issue_idx(g):
            base = g * K
            for ph, (sv, dv) in enumerate(idx_bufs):
                pltpu.make_async_copy(
                    src_hbm.at[pl.ds(ph * PE + base, K)], sv, sem_i).start()
                pltpu.make_async_copy(
                    dst_hbm.at[pl.ds(ph * PE + base, K)], dv, sem_i).start()

        def wait_idx(g):
            base = g * K
            for ph, (sv, dv) in enumerate(idx_bufs):
                pltpu.make_async_copy(
                    src_hbm.at[pl.ds(ph * PE + base, K)], sv, sem_i).wait()
                pltpu.make_async_copy(
                    dst_hbm.at[pl.ds(ph * PE + base, K)], dv, sem_i).wait()

        issue_idx(g0)

        def chunk(g, carry):
            base = g * K
            wait_idx(g)
            for j in range(K // L):
                sl = pl.ds(j * L, L)
                src0[sl] = src0[sl] + coff
                gd0[sl] = dst0[sl] + coff
                src1[sl] = src1[sl] + coff
                gd1[sl] = dst1[sl] + coff
            pltpu.make_async_copy(db_hbm.at[src0], db0, sem_gf).start()
            pltpu.make_async_copy(eh_hbm.at[gd0], ed0, sem_gf).start()
            pltpu.make_async_copy(db_hbm.at[src1], db1, sem_gb).start()
            pltpu.make_async_copy(eh_hbm.at[gd1], ed1, sem_gb).start()
            pltpu.sync_copy(ce_hbm.at[pl.ds(eoff + base, K)], ce_v)

            for ph in range(2):
                db_v = db0 if ph == 0 else db1
                ed_v = ed0 if ph == 0 else ed1
                dst_v = dst0 if ph == 0 else dst1
                sem_g = sem_gf if ph == 0 else sem_gb
                pltpu.make_async_copy(db_hbm.at[src0], db_v, sem_g).wait()
                pltpu.make_async_copy(db_hbm.at[src0], ed_v, sem_g).wait()

                def row(i, rc):
                    for j in range(HALF // L):
                        slp = pl.ds(ph * HALF + j * L, L)
                        sl = pl.ds(j * L, L)
                        sl2 = pl.ds(HALF + j * L, L)
                        en = ce_v[i, slp] + db_v[i, sl] + ed_v[i, sl]
                        if write_enew:
                            ce_v[i, slp] = en
                        sg = 1.0 / (1.0 + jnp.exp(-en))
                        ed_v[i, sl] = sg
                        ed_v[i, sl2] = sg * db_v[i, sl2]
                    return rc

                lax.fori_loop(0, K, row, 0)
                pltpu.make_async_copy(ed_v, acc.at[dst_v], sem_s).start(
                    add=True)
            if write_enew:
                pltpu.sync_copy(ce_v, enew_hbm.at[pl.ds(eoff + base, K)])
            pltpu.make_async_copy(ed0, acc.at[dst0], sem_s).wait()
            pltpu.make_async_copy(ed1, acc.at[dst1], sem_s).wait()

            @pl.when(g + 1 < g1)
            def _():
                issue_idx(g + 1)

            return carry

        lax.fori_loop(g0, g1, chunk, 0)
        plsc.subcore_barrier()
        pltpu.sync_copy(
            acc.at[pl.ds(s * ROWS_PT, ROWS_PT)],
            nd_hbm.at[pl.ds(c * N_ACC + s * ROWS_PT, ROWS_PT)])

    fn = pl.kernel(body, out_type=out_type, mesh=mesh,
                   scratch_types=scratch_types)
    res = fn(ce, src, dst, db, eh)
    return tuple(res) if write_enew else res[0]


def kernel(h, e, edge_index, params):
    src = edge_index[0].astype(jnp.int32)
    dst = edge_index[1].astype(jnp.int32)
    h = _node_emb(h, params["emb_h"])
    e = _edge_emb(e, params["emb_e"])
    layers = params["layers"]
    for l, lp in enumerate(layers):
        last = l == len(layers) - 1
        ah, db, eh = _node_prep(h, lp)
        ce = _ce_packed(
            e, lp["C"]["W"], lp["C"]["b"] + lp["D"]["b"] + lp["E"]["b"],
        ).reshape(2 * PE, DIM)
        db = db.reshape(2 * N_NODES, DIM)
        eh = eh.reshape(2 * N_NODES, DIM)
        if last:
            nd = _sc_message(ce, src, dst, db, eh, write_enew=False)
            return _node_final(ah, nd.reshape(2, N_ACC, DIM), h,
                               lp["bn_h"], lp["B"]["b"], params["mlp"])
        enew, nd = _sc_message(ce, src, dst, db, eh, write_enew=True)
        h = _node_update(ah, nd.reshape(2, N_ACC, DIM), h, lp["bn_h"],
                         lp["B"]["b"])
        enew = enew.reshape(2, PE, DIM)
        st = _e_stats(enew)
        e = _e_update(e, enew, st, lp["bn_e"])
    return None


# speedup vs baseline: 2.9380x; 1.0836x over previous
import jax
import jax.numpy as jnp
from jax import lax
from jax.experimental import pallas as pl
from jax.experimental.pallas import tpu as pltpu
from jax.experimental.pallas import tpu_sc as plsc

N_NODES = 10000
N_EDGES = 320000
PE = N_EDGES // 2
DIM = 128
HALF = 64
EPS_BN = 1e-5
NS = 16
L = 16
K = 64
NCH = PE // K
ROWS_PT = 632
N_ACC = NS * ROWS_PT

EDGE_BLK = 2000
EG = N_EDGES // EDGE_BLK
PG = PE // EDGE_BLK
NODE_BLK = 2000
NG = N_NODES // NODE_BLK

_F32 = jnp.float32


def _mm_kernel(x_ref, w_ref, b_ref, o_ref):
    o_ref[...] = (
        jnp.dot(x_ref[...], w_ref[...], preferred_element_type=_F32)
        + b_ref[...]
    )


def _edge_emb(e, p):
    return pl.pallas_call(
        _mm_kernel,
        grid=(EG,),
        in_specs=[
            pl.BlockSpec((EDGE_BLK, DIM), lambda i: (i, 0)),
            pl.BlockSpec((DIM, DIM), lambda i: (0, 0)),
            pl.BlockSpec((1, DIM), lambda i: (0, 0)),
        ],
        out_specs=pl.BlockSpec((EDGE_BLK, DIM), lambda i: (i, 0)),
        out_shape=jax.ShapeDtypeStruct((N_EDGES, DIM), _F32),
    )(e, p["W"], p["b"].reshape(1, DIM))


def _node_emb(h, p):
    return pl.pallas_call(
        _mm_kernel,
        out_shape=jax.ShapeDtypeStruct((N_NODES, DIM), _F32),
    )(h, p["W"], p["b"].reshape(1, DIM))


def _split_w(w):
    return jnp.stack([w[:, :HALF], w[:, HALF:]])


def _ce_kernel(x1_ref, x2_ref, w_ref, b_ref, o_ref):
    x1 = jnp.dot(x1_ref[...], w_ref[0], preferred_element_type=_F32)
    x2 = jnp.dot(x2_ref[...], w_ref[0], preferred_element_type=_F32)
    o_ref[0] = jnp.concatenate([x1, x2], axis=1) + b_ref[0]


def _ce_packed(e, w, b):
    b_half = b.reshape(2, HALF)
    b2 = jnp.concatenate([b_half, b_half], axis=1).reshape(2, 1, DIM)
    return pl.pallas_call(
        _ce_kernel,
        grid=(PG, 2),
        in_specs=[
            pl.BlockSpec((EDGE_BLK, DIM), lambda i, c: (i, 0)),
            pl.BlockSpec((EDGE_BLK, DIM), lambda i, c: (PG + i, 0)),
            pl.BlockSpec((1, DIM, HALF), lambda i, c: (c, 0, 0)),
            pl.BlockSpec((1, 1, DIM), lambda i, c: (c, 0, 0)),
        ],
        out_specs=pl.BlockSpec((1, EDGE_BLK, DIM), lambda i, c: (c, i, 0)),
        out_shape=jax.ShapeDtypeStruct((2, PE, DIM), _F32),
    )(e, e, _split_w(w), b2)


def _node_prep_kernel(h_ref, wa_ref, ba_ref, wb_ref, wd_ref, we_ref,
                      ah_ref, db_ref, eh_ref):
    h = h_ref[...]
    ah_ref[...] = (
        jnp.dot(h, wa_ref[...], preferred_element_type=_F32) + ba_ref[...]
    )
    bh = jnp.dot(h, wb_ref[0], preferred_element_type=_F32)
    dh = jnp.dot(h, wd_ref[0], preferred_element_type=_F32)
    ehf = jnp.dot(h, we_ref[...], preferred_element_type=_F32)
    db_ref[0] = jnp.concatenate([dh, bh], axis=1)
    c = pl.program_id(1)
    eh_roll = jnp.concatenate([ehf[:, HALF:], ehf[:, :HALF]], axis=1)
    eh_ref[0] = jnp.where(c == 0, ehf, eh_roll)


def _node_prep(h, lp):
    return pl.pallas_call(
        _node_prep_kernel,
        grid=(NG, 2),
        in_specs=[
            pl.BlockSpec((NODE_BLK, DIM), lambda i, c: (i, 0)),
            pl.BlockSpec((DIM, DIM), lambda i, c: (0, 0)),
            pl.BlockSpec((1, DIM), lambda i, c: (0, 0)),
            pl.BlockSpec((1, DIM, HALF), lambda i, c: (c, 0, 0)),
            pl.BlockSpec((1, DIM, HALF), lambda i, c: (c, 0, 0)),
            pl.BlockSpec((DIM, DIM), lambda i, c: (0, 0)),
        ],
        out_specs=[
            pl.BlockSpec((NODE_BLK, DIM), lambda i, c: (i, 0)),
            pl.BlockSpec((1, NODE_BLK, DIM), lambda i, c: (c, i, 0)),
            pl.BlockSpec((1, NODE_BLK, DIM), lambda i, c: (c, i, 0)),
        ],
        out_shape=[
            jax.ShapeDtypeStruct((N_NODES, DIM), _F32),
            jax.ShapeDtypeStruct((2, N_NODES, DIM), _F32),
            jax.ShapeDtypeStruct((2, N_NODES, DIM), _F32),
        ],
    )(h, lp["A"]["W"], lp["A"]["b"].reshape(1, DIM),
      _split_w(lp["B"]["W"]), _split_w(lp["D"]["W"]), lp["E"]["W"])


def _node_update_core(ah_ref, nd_ref, hin_ref, g_ref, b_ref, bb_ref):
    nd = nd_ref[...]
    den = jnp.concatenate(
        [nd[0, :N_NODES, :HALF], nd[1, :N_NODES, :HALF]], axis=1)
    num = jnp.concatenate(
        [nd[0, :N_NODES, HALF:], nd[1, :N_NODES, HALF:]], axis=1)
    x = ah_ref[...] + (num + bb_ref[...] * den) / (den + 1e-6)
    mean = jnp.mean(x, axis=0, keepdims=True)
    var = jnp.mean(x * x, axis=0, keepdims=True) - mean * mean
    xn = g_ref[...] * (x - mean) * lax.rsqrt(var + EPS_BN) + b_ref[...]
    return hin_ref[...] + jnp.maximum(xn, 0.0)


def _node_update_kernel(ah_ref, nd_ref, hin_ref, g_ref, b_ref, bb_ref,
                        o_ref):
    o_ref[...] = _node_update_core(ah_ref, nd_ref, hin_ref, g_ref, b_ref,
                                   bb_ref)


def _node_update(ah, nd, h_in, bn, b_B):
    return pl.pallas_call(
        _node_update_kernel,
        out_shape=jax.ShapeDtypeStruct((N_NODES, DIM), _F32),
    )(ah, nd, h_in, bn["gamma"].reshape(1, DIM), bn["beta"].reshape(1, DIM),
      b_B.reshape(1, DIM))


def _node_final_kernel(ah_ref, nd_ref, hin_ref, g_ref, b_ref, bb_ref,
                       w0_ref, b0_ref, w1_ref, b1_ref, w2_ref, b2_ref,
                       o_ref):
    hout = _node_update_core(ah_ref, nd_ref, hin_ref, g_ref, b_ref, bb_ref)
    hg = jnp.mean(hout, axis=0, keepdims=True)
    y = jnp.maximum(
        jnp.dot(hg, w0_ref[...], preferred_element_type=_F32) + b0_ref[...],
        0.0)
    y = jnp.maximum(
        jnp.dot(y, w1_ref[...], preferred_element_type=_F32) + b1_ref[...],
        0.0)
    o_ref[...] = (
        jnp.dot(y, w2_ref[...], preferred_element_type=_F32) + b2_ref[...]
    )


def _node_final(ah, nd, h_in, bn, b_B, mlp):
    n_classes = mlp[2]["b"].shape[0]
    return pl.pallas_call(
        _node_final_kernel,
        out_shape=jax.ShapeDtypeStruct((1, n_classes), _F32),
    )(ah, nd, h_in, bn["gamma"].reshape(1, DIM), bn["beta"].reshape(1, DIM),
      b_B.reshape(1, DIM),
      mlp[0]["W"], mlp[0]["b"].reshape(1, -1),
      mlp[1]["W"], mlp[1]["b"].reshape(1, -1),
      mlp[2]["W"], mlp[2]["b"].reshape(1, -1))


def _e_stats_kernel(e0_ref, e1_ref, o_ref):
    i = pl.program_id(0)
    b0 = e0_ref[0]
    b1 = e1_ref[0]
    s0 = jnp.sum(b0[:, :HALF], axis=0) + jnp.sum(b0[:, HALF:], axis=0)
    s1 = jnp.sum(b1[:, :HALF], axis=0) + jnp.sum(b1[:, HALF:], axis=0)
    q0 = (jnp.sum(b0[:, :HALF] * b0[:, :HALF], axis=0)
          + jnp.sum(b0[:, HALF:] * b0[:, HALF:], axis=0))
    q1 = (jnp.sum(b1[:, :HALF] * b1[:, :HALF], axis=0)
          + jnp.sum(b1[:, HALF:] * b1[:, HALF:], axis=0))
    blk = jnp.stack([jnp.concatenate([s0, s1]), jnp.concatenate([q0, q1])])

    @pl.when(i == 0)
    def _():
        o_ref[...] = blk

    @pl.when(i > 0)
    def _():
        o_ref[...] += blk


def _e_stats(enew):
    return pl.pallas_call(
        _e_stats_kernel,
        grid=(PG,),
        in_specs=[
            pl.BlockSpec((1, EDGE_BLK, DIM), lambda i: (0, i, 0)),
            pl.BlockSpec((1, EDGE_BLK, DIM), lambda i: (1, i, 0)),
        ],
        out_specs=pl.BlockSpec((2, DIM), lambda i: (0, 0)),
        out_shape=jax.ShapeDtypeStruct((2, DIM), _F32),
    )(enew, enew)


def _e_update_kernel(ein_ref, e0_ref, e1_ref, st_ref, g_ref, b_ref, o_ref):
    front = pl.program_id(0) < PG
    b0 = e0_ref[0]
    b1 = e1_ref[0]
    en_c0 = jnp.where(front, b0[:, :HALF], b0[:, HALF:])
    en_c1 = jnp.where(front, b1[:, :HALF], b1[:, HALF:])
    en = jnp.concatenate([en_c0, en_c1], axis=1)
    st = st_ref[...]
    mean = st[0:1] / N_EDGES
    var = st[1:2] / N_EDGES - mean * mean
    xn = g_ref[...] * (en - mean) * lax.rsqrt(var + EPS_BN) + b_ref[...]
    o_ref[...] = ein_ref[...] + jnp.maximum(xn, 0.0)


def _e_update(e_in, enew, st, bn):
    return pl.pallas_call(
        _e_update_kernel,
        grid=(EG,),
        in_specs=[
            pl.BlockSpec((EDGE_BLK, DIM), lambda i: (i, 0)),
            pl.BlockSpec((1, EDGE_BLK, DIM), lambda i: (0, i % PG, 0)),
            pl.BlockSpec((1, EDGE_BLK, DIM), lambda i: (1, i % PG, 0)),
            pl.BlockSpec((2, DIM), lambda i: (0, 0)),
            pl.BlockSpec((1, DIM), lambda i: (0, 0)),
            pl.BlockSpec((1, DIM), lambda i: (0, 0)),
        ],
        out_specs=pl.BlockSpec((EDGE_BLK, DIM), lambda i: (i, 0)),
        out_shape=jax.ShapeDtypeStruct((N_EDGES, DIM), _F32),
    )(e_in, enew, enew, st, bn["gamma"].reshape(1, DIM),
      bn["beta"].reshape(1, DIM))


def _ce_fused_kernel(write_e, ef_ref, eb_ref, en0_ref, en1_ref, st_ref,
                     g_ref, b_ref, w_ref, b2_ref, *out_refs):
    b0 = en0_ref[0]
    b1 = en1_ref[0]
    en_f = jnp.concatenate([b0[:, :HALF], b1[:, :HALF]], axis=1)
    en_b = jnp.concatenate([b0[:, HALF:], b1[:, HALF:]], axis=1)
    st = st_ref[...]
    mean = st[0:1] / N_EDGES
    var = st[1:2] / N_EDGES - mean * mean
    rstd = lax.rsqrt(var + EPS_BN)
    g = g_ref[...]
    b = b_ref[...]
    e_f = ef_ref[...] + jnp.maximum(g * (en_f - mean) * rstd + b, 0.0)
    e_b = eb_ref[...] + jnp.maximum(g * (en_b - mean) * rstd + b, 0.0)
    x_f = jnp.dot(e_f, w_ref[0], preferred_element_type=_F32)
    x_b = jnp.dot(e_b, w_ref[0], preferred_element_type=_F32)
    out_refs[0][0] = jnp.concatenate([x_f, x_b], axis=1) + b2_ref[0]
    if write_e:
        out_refs[1][...] = e_f
        out_refs[2][...] = e_b


def _ce_fused(e_f, e_b, first, enew, st, bn, w, bias, write_e):
    import functools
    b_half = bias.reshape(2, HALF)
    b2 = jnp.concatenate([b_half, b_half], axis=1).reshape(2, 1, DIM)
    if first:
        ef_spec = pl.BlockSpec((EDGE_BLK, DIM), lambda i, c: (i, 0))
        eb_spec = pl.BlockSpec((EDGE_BLK, DIM), lambda i, c: (PG + i, 0))
    else:
        ef_spec = pl.BlockSpec((EDGE_BLK, DIM), lambda i, c: (i, 0))
        eb_spec = pl.BlockSpec((EDGE_BLK, DIM), lambda i, c: (i, 0))
    out_specs = [pl.BlockSpec((1, EDGE_BLK, DIM), lambda i, c: (c, i, 0))]
    out_shape = [jax.ShapeDtypeStruct((2, PE, DIM), _F32)]
    if write_e:
        out_specs += [
            pl.BlockSpec((EDGE_BLK, DIM), lambda i, c: (i, 0)),
            pl.BlockSpec((EDGE_BLK, DIM), lambda i, c: (i, 0)),
        ]
        out_shape += [
            jax.ShapeDtypeStruct((PE, DIM), _F32),
            jax.ShapeDtypeStruct((PE, DIM), _F32),
        ]
    return pl.pallas_call(
        functools.partial(_ce_fused_kernel, write_e),
        grid=(PG, 2),
        in_specs=[
            ef_spec,
            eb_spec,
            pl.BlockSpec((1, EDGE_BLK, DIM), lambda i, c: (0, i, 0)),
            pl.BlockSpec((1, EDGE_BLK, DIM), lambda i, c: (1, i, 0)),
            pl.BlockSpec((2, DIM), lambda i, c: (0, 0)),
            pl.BlockSpec((1, DIM), lambda i, c: (0, 0)),
            pl.BlockSpec((1, DIM), lambda i, c: (0, 0)),
            pl.BlockSpec((1, DIM, HALF), lambda i, c: (c, 0, 0)),
            pl.BlockSpec((1, 1, DIM), lambda i, c: (c, 0, 0)),
        ],
        out_specs=out_specs,
        out_shape=out_shape,
    )(e_f, e_b, enew, enew, st, bn["gamma"].reshape(1, DIM),
      bn["beta"].reshape(1, DIM), _split_w(w), b2)


def _sc_message(ce, src, dst, db, eh, write_enew):
    mesh = plsc.VectorSubcoreMesh(core_axis_name="c", subcore_axis_name="s")
    out_type = []
    if write_enew:
        out_type.append(jax.ShapeDtypeStruct((2 * PE, DIM), _F32))
    out_type.append(jax.ShapeDtypeStruct((2 * N_ACC, DIM), _F32))
    scratch_types = [
        pltpu.VMEM((K,), jnp.int32),
        pltpu.VMEM((K,), jnp.int32),
        pltpu.VMEM((K,), jnp.int32),
        pltpu.VMEM((K,), jnp.int32),
        pltpu.VMEM((K,), jnp.int32),
        pltpu.VMEM((K,), jnp.int32),
        pltpu.VMEM((K, DIM), _F32),
        pltpu.VMEM((K, DIM), _F32),
        pltpu.VMEM((K, DIM), _F32),
        pltpu.VMEM((K, DIM), _F32),
        pltpu.VMEM((K, DIM), _F32),
        pltpu.VMEM_SHARED((N_ACC, DIM), _F32),
        pltpu.SemaphoreType.DMA,
        pltpu.SemaphoreType.DMA,
        pltpu.SemaphoreType.DMA,
        pltpu.SemaphoreType.DMA,
    ]

    def body(ce_hbm, src_hbm, dst_hbm, db_hbm, eh_hbm, *refs):
        if write_enew:
            enew_hbm, nd_hbm = refs[0], refs[1]
            scr = refs[2:]
        else:
            enew_hbm, nd_hbm = None, refs[0]
            scr = refs[1:]
        (src0, dst0, gd0, src1, dst1, gd1, ce_v, db0, ed0, db1, ed1,
         acc, sem_gf, sem_gb, sem_s, sem_i) = scr
        c = lax.axis_index("c")
        s = lax.axis_index("s")

        @pl.loop(0, K)
        def _(i):
            for j in range(DIM // L):
                ed0[i, pl.ds(j * L, L)] = jnp.zeros((L,), _F32)

        for t in range(ROWS_PT // K):
            pltpu.sync_copy(ed0, acc.at[pl.ds(s * ROWS_PT + t * K, K)])
        _rem = ROWS_PT % K
        if _rem:
            pltpu.sync_copy(
                ed0.at[pl.ds(0, _rem)],
                acc.at[pl.ds(s * ROWS_PT + (ROWS_PT // K) * K, _rem)])
        plsc.subcore_barrier()

        coff = c * N_NODES
        eoff = c * PE
        g0 = (NCH * s) // NS
        g1 = (NCH * (s + 1)) // NS

        idx_bufs = ((src0, dst0), (src1, dst1))

        def issue_idx(g):
            base = g * K
            for ph, (sv, dv) in enumerate(idx_bufs):
                pltpu.make_async_copy(
                    src_hbm.at[pl.ds(ph * PE + base, K)], sv, sem_i).start()
                pltpu.make_async_copy(
                    dst_hbm.at[pl.ds(ph * PE + base, K)], dv, sem_i).start()

        def wait_idx(g):
            base = g * K
            for ph, (sv, dv) in enumerate(idx_bufs):
                pltpu.make_async_copy(
                    src_hbm.at[pl.ds(ph * PE + base, K)], sv, sem_i).wait()
                pltpu.make_async_copy(
                    dst_hbm.at[pl.ds(ph * PE + base, K)], dv, sem_i).wait()

        issue_idx(g0)

        def chunk(g, carry):
            base = g * K
            wait_idx(g)
            for j in range(K // L):
                sl = pl.ds(j * L, L)
                src0[sl] = src0[sl] + coff
                gd0[sl] = dst0[sl] + coff
                src1[sl] = src1[sl] + coff
                gd1[sl] = dst1[sl] + coff
            pltpu.make_async_copy(db_hbm.at[src0], db0, sem_gf).start()
            pltpu.make_async_copy(eh_hbm.at[gd0], ed0, sem_gf).start()
            pltpu.make_async_copy(db_hbm.at[src1], db1, sem_gb).start()
            pltpu.make_async_copy(eh_hbm.at[gd1], ed1, sem_gb).start()
            pltpu.sync_copy(ce_hbm.at[pl.ds(eoff + base, K)], ce_v)

            for ph in range(2):
                db_v = db0 if ph == 0 else db1
                ed_v = ed0 if ph == 0 else ed1
                dst_v = dst0 if ph == 0 else dst1
                sem_g = sem_gf if ph == 0 else sem_gb
                pltpu.make_async_copy(db_hbm.at[src0], db_v, sem_g).wait()
                pltpu.make_async_copy(db_hbm.at[src0], ed_v, sem_g).wait()

                def row(i, rc):
                    for j in range(HALF // L):
                        slp = pl.ds(ph * HALF + j * L, L)
                        sl = pl.ds(j * L, L)
                        sl2 = pl.ds(HALF + j * L, L)
                        en = ce_v[i, slp] + db_v[i, sl] + ed_v[i, sl]
                        if write_enew:
                            ce_v[i, slp] = en
                        sg = 1.0 / (1.0 + jnp.exp(-en))
                        ed_v[i, sl] = sg
                        ed_v[i, sl2] = sg * db_v[i, sl2]
                    return rc

                lax.fori_loop(0, K, row, 0)
                pltpu.make_async_copy(ed_v, acc.at[dst_v], sem_s).start(
                    add=True)
            if write_enew:
                pltpu.sync_copy(ce_v, enew_hbm.at[pl.ds(eoff + base, K)])
            pltpu.make_async_copy(ed0, acc.at[dst0], sem_s).wait()
            pltpu.make_async_copy(ed1, acc.at[dst1], sem_s).wait()

            @pl.when(g + 1 < g1)
            def _():
                issue_idx(g + 1)

            return carry

        lax.fori_loop(g0, g1, chunk, 0)
        plsc.subcore_barrier()
        pltpu.sync_copy(
            acc.at[pl.ds(s * ROWS_PT, ROWS_PT)],
            nd_hbm.at[pl.ds(c * N_ACC + s * ROWS_PT, ROWS_PT)])

    fn = pl.kernel(body, out_type=out_type, mesh=mesh,
                   scratch_types=scratch_types)
    res = fn(ce, src, dst, db, eh)
    return tuple(res) if write_enew else res[0]


def kernel(h, e, edge_index, params):
    src = edge_index[0].astype(jnp.int32)
    dst = edge_index[1].astype(jnp.int32)
    h = _node_emb(h, params["emb_h"])
    e0 = _edge_emb(e, params["emb_e"])
    layers = params["layers"]
    e_f = e_b = None
    enew = st = prev_bn = None
    y = None
    for l, lp in enumerate(layers):
        last = l == len(layers) - 1
        ah, db, eh = _node_prep(h, lp)
        bias = lp["C"]["b"] + lp["D"]["b"] + lp["E"]["b"]
        if l == 0:
            ce = _ce_packed(e0, lp["C"]["W"], bias)
        elif l == 1:
            res = _ce_fused(e0, e0, True, enew, st, prev_bn,
                            lp["C"]["W"], bias, write_e=True)
            ce, e_f, e_b = res
        else:
            res = _ce_fused(e_f, e_b, False, enew, st, prev_bn,
                            lp["C"]["W"], bias, write_e=not last)
            if last:
                ce = res[0]
            else:
                ce, e_f, e_b = res
        ce = ce.reshape(2 * PE, DIM)
        db = db.reshape(2 * N_NODES, DIM)
        eh = eh.reshape(2 * N_NODES, DIM)
        if last:
            nd = _sc_message(ce, src, dst, db, eh, write_enew=False)
            y = _node_final(ah, nd.reshape(2, N_ACC, DIM), h,
                            lp["bn_h"], lp["B"]["b"], params["mlp"])
        else:
            enew, nd = _sc_message(ce, src, dst, db, eh, write_enew=True)
            h = _node_update(ah, nd.reshape(2, N_ACC, DIM), h, lp["bn_h"],
                             lp["B"]["b"])
            enew = enew.reshape(2, PE, DIM)
            st = _e_stats(enew)
            prev_bn = lp["bn_e"]
    return y


# speedup vs baseline: 2.9851x; 1.0160x over previous
import dataclasses
import jax
import jax.numpy as jnp
import numpy as np
from jax import lax
from jax.experimental import pallas as pl
from jax.experimental.pallas import tpu as pltpu
from jax.experimental.pallas import tpu_sc as plsc

N_NODES = 10000
N_EDGES = 320000
PE = N_EDGES // 2
DIM = 128
HALF = 64
EPS_BN = 1e-5
NS = 16
L = 16
K = 64
NCH = PE // K
ROWS_PT = 632
N_ACC = NS * ROWS_PT

EDGE_BLK = 2000
EG = N_EDGES // EDGE_BLK
PG = PE // EDGE_BLK
NODE_BLK = 2000
NG = N_NODES // NODE_BLK

_F32 = jnp.float32
_BF16 = jnp.bfloat16


def _bdot(x, w):
    return jnp.dot(x.astype(_BF16), w.astype(_BF16),
                   preferred_element_type=_F32)


def _mm_kernel(x_ref, w_ref, b_ref, o_ref):
    o_ref[...] = _bdot(x_ref[...], w_ref[...]) + b_ref[...]


def _edge_emb(e, p):
    return pl.pallas_call(
        _mm_kernel,
        grid=(EG,),
        in_specs=[
            pl.BlockSpec((EDGE_BLK, DIM), lambda i: (i, 0)),
            pl.BlockSpec((DIM, DIM), lambda i: (0, 0)),
            pl.BlockSpec((1, DIM), lambda i: (0, 0)),
        ],
        out_specs=pl.BlockSpec((EDGE_BLK, DIM), lambda i: (i, 0)),
        out_shape=jax.ShapeDtypeStruct((N_EDGES, DIM), _F32),
    )(e, p["W"], p["b"].reshape(1, DIM))


def _node_emb(h, p):
    return pl.pallas_call(
        _mm_kernel,
        out_shape=jax.ShapeDtypeStruct((N_NODES, DIM), _F32),
    )(h, p["W"], p["b"].reshape(1, DIM))


def _split_w(w):
    return jnp.stack([w[:, :HALF], w[:, HALF:]])


def _ce_kernel(x1_ref, x2_ref, w_ref, b_ref, o_ref):
    x1 = _bdot(x1_ref[...], w_ref[0])
    x2 = _bdot(x2_ref[...], w_ref[0])
    o_ref[0] = jnp.concatenate([x1, x2], axis=1) + b_ref[0]


def _ce_packed(e, w, b):
    b_half = b.reshape(2, HALF)
    b2 = jnp.concatenate([b_half, b_half], axis=1).reshape(2, 1, DIM)
    return pl.pallas_call(
        _ce_kernel,
        grid=(PG, 2),
        in_specs=[
            pl.BlockSpec((EDGE_BLK, DIM), lambda i, c: (i, 0)),
            pl.BlockSpec((EDGE_BLK, DIM), lambda i, c: (PG + i, 0)),
            pl.BlockSpec((1, DIM, HALF), lambda i, c: (c, 0, 0)),
            pl.BlockSpec((1, 1, DIM), lambda i, c: (c, 0, 0)),
        ],
        out_specs=pl.BlockSpec((1, EDGE_BLK, DIM), lambda i, c: (c, i, 0)),
        out_shape=jax.ShapeDtypeStruct((2, PE, DIM), _F32),
    )(e, e, _split_w(w), b2)


def _emb_ce_kernel(x1_ref, x2_ref, we_ref, be_ref, w_ref, b2_ref,
                   ce_ref, ef_ref, eb_ref):
    e_f = _bdot(x1_ref[...], we_ref[...]) + be_ref[...]
    e_b = _bdot(x2_ref[...], we_ref[...]) + be_ref[...]
    x1 = _bdot(e_f, w_ref[0])
    x2 = _bdot(e_b, w_ref[0])
    ce_ref[0] = jnp.concatenate([x1, x2], axis=1) + b2_ref[0]
    ef_ref[...] = e_f
    eb_ref[...] = e_b


def _emb_ce(e, pe, w, bias):
    b_half = bias.reshape(2, HALF)
    b2 = jnp.concatenate([b_half, b_half], axis=1).reshape(2, 1, DIM)
    return pl.pallas_call(
        _emb_ce_kernel,
        grid=(PG, 2),
        in_specs=[
            pl.BlockSpec((EDGE_BLK, DIM), lambda i, c: (i, 0)),
            pl.BlockSpec((EDGE_BLK, DIM), lambda i, c: (PG + i, 0)),
            pl.BlockSpec((DIM, DIM), lambda i, c: (0, 0)),
            pl.BlockSpec((1, DIM), lambda i, c: (0, 0)),
            pl.BlockSpec((1, DIM, HALF), lambda i, c: (c, 0, 0)),
            pl.BlockSpec((1, 1, DIM), lambda i, c: (c, 0, 0)),
        ],
        out_specs=[
            pl.BlockSpec((1, EDGE_BLK, DIM), lambda i, c: (c, i, 0)),
            pl.BlockSpec((EDGE_BLK, DIM), lambda i, c: (i, 0)),
            pl.BlockSpec((EDGE_BLK, DIM), lambda i, c: (i, 0)),
        ],
        out_shape=[
            jax.ShapeDtypeStruct((2, PE, DIM), _F32),
            jax.ShapeDtypeStruct((PE, DIM), _F32),
            jax.ShapeDtypeStruct((PE, DIM), _F32),
        ],
    )(e, e, pe["W"], pe["b"].reshape(1, DIM), _split_w(w), b2)


def _node_prep_kernel(h_ref, wa_ref, ba_ref, wb_ref, wd_ref, we_ref,
                      ah_ref, db_ref, eh_ref):
    h = h_ref[...]
    ah_ref[...] = (
        jnp.dot(h, wa_ref[...], preferred_element_type=_F32) + ba_ref[...]
    )
    bh = jnp.dot(h, wb_ref[0], preferred_element_type=_F32)
    dh = jnp.dot(h, wd_ref[0], preferred_element_type=_F32)
    ehc = jnp.dot(h, we_ref[0], preferred_element_type=_F32)
    db_ref[0] = jnp.concatenate([dh, bh], axis=1)
    eh_ref[0] = jnp.concatenate([ehc, ehc], axis=1)


def _node_prep(h, lp):
    return pl.pallas_call(
        _node_prep_kernel,
        grid=(NG, 2),
        in_specs=[
            pl.BlockSpec((NODE_BLK, DIM), lambda i, c: (i, 0)),
            pl.BlockSpec((DIM, DIM), lambda i, c: (0, 0)),
            pl.BlockSpec((1, DIM), lambda i, c: (0, 0)),
            pl.BlockSpec((1, DIM, HALF), lambda i, c: (c, 0, 0)),
            pl.BlockSpec((1, DIM, HALF), lambda i, c: (c, 0, 0)),
            pl.BlockSpec((1, DIM, HALF), lambda i, c: (c, 0, 0)),
        ],
        out_specs=[
            pl.BlockSpec((NODE_BLK, DIM), lambda i, c: (i, 0)),
            pl.BlockSpec((1, NODE_BLK, DIM), lambda i, c: (c, i, 0)),
            pl.BlockSpec((1, NODE_BLK, DIM), lambda i, c: (c, i, 0)),
        ],
        out_shape=[
            jax.ShapeDtypeStruct((N_NODES, DIM), _F32),
            jax.ShapeDtypeStruct((2, N_NODES, DIM), _F32),
            jax.ShapeDtypeStruct((2, N_NODES, DIM), _F32),
        ],
    )(h, lp["A"]["W"], lp["A"]["b"].reshape(1, DIM),
      _split_w(lp["B"]["W"]), _split_w(lp["D"]["W"]),
      _split_w(lp["E"]["W"]))


def _node_update_core(ah_ref, nd_ref, hin_ref, g_ref, b_ref, bb_ref):
    nd = nd_ref[...]
    den = jnp.concatenate(
        [nd[0, :N_NODES, :HALF], nd[1, :N_NODES, :HALF]], axis=1)
    num = jnp.concatenate(
        [nd[0, :N_NODES, HALF:], nd[1, :N_NODES, HALF:]], axis=1)
    x = ah_ref[...] + (num + bb_ref[...] * den) / (den + 1e-6)
    mean = jnp.mean(x, axis=0, keepdims=True)
    var = jnp.mean(x * x, axis=0, keepdims=True) - mean * mean
    xn = g_ref[...] * (x - mean) * lax.rsqrt(var + EPS_BN) + b_ref[...]
    return hin_ref[...] + jnp.maximum(xn, 0.0)


def _node_update_kernel(ah_ref, nd_ref, hin_ref, g_ref, b_ref, bb_ref,
                        o_ref):
    o_ref[...] = _node_update_core(ah_ref, nd_ref, hin_ref, g_ref, b_ref,
                                   bb_ref)


def _node_update(ah, nd, h_in, bn, b_B):
    return pl.pallas_call(
        _node_update_kernel,
        out_shape=jax.ShapeDtypeStruct((N_NODES, DIM), _F32),
    )(ah, nd, h_in, bn["gamma"].reshape(1, DIM), bn["beta"].reshape(1, DIM),
      b_B.reshape(1, DIM))


def _node_final_kernel(ah_ref, nd_ref, hin_ref, g_ref, b_ref, bb_ref,
                       w0_ref, b0_ref, w1_ref, b1_ref, w2_ref, b2_ref,
                       o_ref):
    hout = _node_update_core(ah_ref, nd_ref, hin_ref, g_ref, b_ref, bb_ref)
    hg = jnp.mean(hout, axis=0, keepdims=True)
    y = jnp.maximum(
        jnp.dot(hg, w0_ref[...], preferred_element_type=_F32) + b0_ref[...],
        0.0)
    y = jnp.maximum(
        jnp.dot(y, w1_ref[...], preferred_element_type=_F32) + b1_ref[...],
        0.0)
    o_ref[...] = (
        jnp.dot(y, w2_ref[...], preferred_element_type=_F32) + b2_ref[...]
    )


def _node_final(ah, nd, h_in, bn, b_B, mlp):
    n_classes = mlp[2]["b"].shape[0]
    return pl.pallas_call(
        _node_final_kernel,
        out_shape=jax.ShapeDtypeStruct((1, n_classes), _F32),
    )(ah, nd, h_in, bn["gamma"].reshape(1, DIM), bn["beta"].reshape(1, DIM),
      b_B.reshape(1, DIM),
      mlp[0]["W"], mlp[0]["b"].reshape(1, -1),
      mlp[1]["W"], mlp[1]["b"].reshape(1, -1),
      mlp[2]["W"], mlp[2]["b"].reshape(1, -1))


def _e_stats_kernel(e0_ref, e1_ref, o_ref):
    i = pl.program_id(0)
    b0 = e0_ref[0]
    b1 = e1_ref[0]
    s0 = jnp.sum(b0[:, :HALF], axis=0) + jnp.sum(b0[:, HALF:], axis=0)
    s1 = jnp.sum(b1[:, :HALF], axis=0) + jnp.sum(b1[:, HALF:], axis=0)
    q0 = (jnp.sum(b0[:, :HALF] * b0[:, :HALF], axis=0)
          + jnp.sum(b0[:, HALF:] * b0[:, HALF:], axis=0))
    q1 = (jnp.sum(b1[:, :HALF] * b1[:, :HALF], axis=0)
          + jnp.sum(b1[:, HALF:] * b1[:, HALF:], axis=0))
    blk = jnp.stack([jnp.concatenate([s0, s1]), jnp.concatenate([q0, q1])])

    @pl.when(i == 0)
    def _():
        o_ref[...] = blk

    @pl.when(i > 0)
    def _():
        o_ref[...] += blk


def _e_stats(enew):
    return pl.pallas_call(
        _e_stats_kernel,
        grid=(PG,),
        in_specs=[
            pl.BlockSpec((1, EDGE_BLK, DIM), lambda i: (0, i, 0)),
            pl.BlockSpec((1, EDGE_BLK, DIM), lambda i: (1, i, 0)),
        ],
        out_specs=pl.BlockSpec((2, DIM), lambda i: (0, 0)),
        out_shape=jax.ShapeDtypeStruct((2, DIM), _F32),
    )(enew, enew)


def _e_update_kernel(ein_ref, e0_ref, e1_ref, st_ref, g_ref, b_ref, o_ref):
    front = pl.program_id(0) < PG
    b0 = e0_ref[0]
    b1 = e1_ref[0]
    en_c0 = jnp.where(front, b0[:, :HALF], b0[:, HALF:])
    en_c1 = jnp.where(front, b1[:, :HALF], b1[:, HALF:])
    en = jnp.concatenate([en_c0, en_c1], axis=1)
    st = st_ref[...]
    mean = st[0:1] / N_EDGES
    var = st[1:2] / N_EDGES - mean * mean
    xn = g_ref[...] * (en - mean) * lax.rsqrt(var + EPS_BN) + b_ref[...]
    o_ref[...] = ein_ref[...] + jnp.maximum(xn, 0.0)


def _e_update(e_in, enew, st, bn):
    return pl.pallas_call(
        _e_update_kernel,
        grid=(EG,),
        in_specs=[
            pl.BlockSpec((EDGE_BLK, DIM), lambda i: (i, 0)),
            pl.BlockSpec((1, EDGE_BLK, DIM), lambda i: (0, i % PG, 0)),
            pl.BlockSpec((1, EDGE_BLK, DIM), lambda i: (1, i % PG, 0)),
            pl.BlockSpec((2, DIM), lambda i: (0, 0)),
            pl.BlockSpec((1, DIM), lambda i: (0, 0)),
            pl.BlockSpec((1, DIM), lambda i: (0, 0)),
        ],
        out_specs=pl.BlockSpec((EDGE_BLK, DIM), lambda i: (i, 0)),
        out_shape=jax.ShapeDtypeStruct((N_EDGES, DIM), _F32),
    )(e_in, enew, enew, st, bn["gamma"].reshape(1, DIM),
      bn["beta"].reshape(1, DIM))


def _ce_fused_kernel(write_e, ef_ref, eb_ref, en0_ref, en1_ref, st_ref,
                     g_ref, b_ref, w_ref, b2_ref, *out_refs):
    b0 = en0_ref[0]
    b1 = en1_ref[0]
    en_f = jnp.concatenate([b0[:, :HALF], b1[:, :HALF]], axis=1)
    en_b = jnp.concatenate([b0[:, HALF:], b1[:, HALF:]], axis=1)
    st = st_ref[...]
    mean = st[0:1] / N_EDGES
    var = st[1:2] / N_EDGES - mean * mean
    rstd = lax.rsqrt(var + EPS_BN)
    g = g_ref[...]
    b = b_ref[...]
    e_f = ef_ref[...] + jnp.maximum(g * (en_f - mean) * rstd + b, 0.0)
    e_b = eb_ref[...] + jnp.maximum(g * (en_b - mean) * rstd + b, 0.0)
    x_f = _bdot(e_f, w_ref[0])
    x_b = _bdot(e_b, w_ref[0])
    out_refs[0][0] = jnp.concatenate([x_f, x_b], axis=1) + b2_ref[0]
    if write_e:
        out_refs[1][...] = e_f
        out_refs[2][...] = e_b


def _ce_fused(e_f, e_b, first, enew, st, bn, w, bias, write_e):
    import functools
    b_half = bias.reshape(2, HALF)
    b2 = jnp.concatenate([b_half, b_half], axis=1).reshape(2, 1, DIM)
    if first:
        ef_spec = pl.BlockSpec((EDGE_BLK, DIM), lambda i, c: (i, 0))
        eb_spec = pl.BlockSpec((EDGE_BLK, DIM), lambda i, c: (PG + i, 0))
    else:
        ef_spec = pl.BlockSpec((EDGE_BLK, DIM), lambda i, c: (i, 0))
        eb_spec = pl.BlockSpec((EDGE_BLK, DIM), lambda i, c: (i, 0))
    out_specs = [pl.BlockSpec((1, EDGE_BLK, DIM), lambda i, c: (c, i, 0))]
    out_shape = [jax.ShapeDtypeStruct((2, PE, DIM), _F32)]
    if write_e:
        out_specs += [
            pl.BlockSpec((EDGE_BLK, DIM), lambda i, c: (i, 0)),
            pl.BlockSpec((EDGE_BLK, DIM), lambda i, c: (i, 0)),
        ]
        out_shape += [
            jax.ShapeDtypeStruct((PE, DIM), _F32),
            jax.ShapeDtypeStruct((PE, DIM), _F32),
        ]
    return pl.pallas_call(
        functools.partial(_ce_fused_kernel, write_e),
        grid=(PG, 2),
        in_specs=[
            ef_spec,
            eb_spec,
            pl.BlockSpec((1, EDGE_BLK, DIM), lambda i, c: (0, i, 0)),
            pl.BlockSpec((1, EDGE_BLK, DIM), lambda i, c: (1, i, 0)),
            pl.BlockSpec((2, DIM), lambda i, c: (0, 0)),
            pl.BlockSpec((1, DIM), lambda i, c: (0, 0)),
            pl.BlockSpec((1, DIM), lambda i, c: (0, 0)),
            pl.BlockSpec((1, DIM, HALF), lambda i, c: (c, 0, 0)),
            pl.BlockSpec((1, 1, DIM), lambda i, c: (c, 0, 0)),
        ],
        out_specs=out_specs,
        out_shape=out_shape,
    )(e_f, e_b, enew, enew, st, bn["gamma"].reshape(1, DIM),
      bn["beta"].reshape(1, DIM), _split_w(w), b2)


def _sc_message(ce, src, dst, db, eh, write_enew):
    mesh = plsc.VectorSubcoreMesh(core_axis_name="c", subcore_axis_name="s")
    out_type = []
    if write_enew:
        out_type.append(jax.ShapeDtypeStruct((2 * PE, DIM), _F32))
    out_type.append(jax.ShapeDtypeStruct((2 * N_ACC, DIM), _F32))
    scratch_types = [
        pltpu.VMEM((K,), jnp.int32),
        pltpu.VMEM((K,), jnp.int32),
        pltpu.VMEM((K,), jnp.int32),
        pltpu.VMEM((K,), jnp.int32),
        pltpu.VMEM((K,), jnp.int32),
        pltpu.VMEM((K,), jnp.int32),
        pltpu.VMEM((K, DIM), _F32),
        pltpu.VMEM((K, DIM), _F32),
        pltpu.VMEM((K, DIM), _F32),
        pltpu.VMEM((K, DIM), _F32),
        pltpu.VMEM((K, DIM), _F32),
        pltpu.VMEM_SHARED((N_ACC, DIM), _F32),
        pltpu.SemaphoreType.DMA,
        pltpu.SemaphoreType.DMA,
        pltpu.SemaphoreType.DMA,
        pltpu.SemaphoreType.DMA,
    ]

    def body(ce_hbm, src_hbm, dst_hbm, db_hbm, eh_hbm, *refs):
        if write_enew:
            enew_hbm, nd_hbm = refs[0], refs[1]
            scr = refs[2:]
        else:
            enew_hbm, nd_hbm = None, refs[0]
            scr = refs[1:]
        (src0, dst0, gd0, src1, dst1, gd1, ce_v, db0, ed0, db1, ed1,
         acc, sem_gf, sem_gb, sem_s, sem_i) = scr
        c = lax.axis_index("c")
        s = lax.axis_index("s")

        @pl.loop(0, K)
        def _(i):
            for j in range(DIM // L):
                ed0[i, pl.ds(j * L, L)] = jnp.zeros((L,), _F32)

        for t in range(ROWS_PT // K):
            pltpu.sync_copy(ed0, acc.at[pl.ds(s * ROWS_PT + t * K, K)])
        _rem = ROWS_PT % K
        if _rem:
            pltpu.sync_copy(
                ed0.at[pl.ds(0, _rem)],
                acc.at[pl.ds(s * ROWS_PT + (ROWS_PT // K) * K, _rem)])
        plsc.subcore_barrier()

        coff = c * N_NODES
        eoff = c * PE
        g0 = (NCH * s) // NS
        g1 = (NCH * (s + 1)) // NS

        idx_bufs = ((src0, dst0), (src1, dst1))

        def issue_idx(g):
            base = g * K
            for ph, (sv, dv) in enumerate(idx_bufs):
                pltpu.make_async_copy(
                    src_hbm.at[pl.ds(ph * PE + base, K)], sv, sem_i).start()
                pltpu.make_async_copy(
                    dst_hbm.at[pl.ds(ph * PE + base, K)], dv, sem_i).start()

        def wait_idx(g):
            base = g * K
            for ph, (sv, dv) in enumerate(idx_bufs):
                pltpu.make_async_copy(
                    src_hbm.at[pl.ds(ph * PE + base, K)], sv, sem_i).wait()
                pltpu.make_async_copy(
                    dst_hbm.at[pl.ds(ph * PE + base, K)], dv, sem_i).wait()

        issue_idx(g0)

        def chunk(g, carry):
            base = g * K
            wait_idx(g)
            for j in range(K // L):
                sl = pl.ds(j * L, L)
                src0[sl] = src0[sl] + coff
                gd0[sl] = dst0[sl] + coff
                src1[sl] = src1[sl] + coff
                gd1[sl] = dst1[sl] + coff
            pltpu.make_async_copy(db_hbm.at[src0], db0, sem_gf).start()
            pltpu.make_async_copy(eh_hbm.at[gd0], ed0, sem_gf).start()
            pltpu.make_async_copy(db_hbm.at[src1], db1, sem_gb).start()
            pltpu.make_async_copy(eh_hbm.at[gd1], ed1, sem_gb).start()
            pltpu.sync_copy(ce_hbm.at[pl.ds(eoff + base, K)], ce_v)

            for ph in range(2):
                db_v = db0 if ph == 0 else db1
                ed_v = ed0 if ph == 0 else ed1
                dst_v = dst0 if ph == 0 else dst1
                sem_g = sem_gf if ph == 0 else sem_gb
                pltpu.make_async_copy(db_hbm.at[src0], db_v, sem_g).wait()
                pltpu.make_async_copy(db_hbm.at[src0], ed_v, sem_g).wait()

                def row(i, rc):
                    for j in range(HALF // L):
                        slp = pl.ds(ph * HALF + j * L, L)
                        sl = pl.ds(j * L, L)
                        sl2 = pl.ds(HALF + j * L, L)
                        en = ce_v[i, slp] + db_v[i, sl] + ed_v[i, sl]
                        if write_enew:
                            ce_v[i, slp] = en
                        sg = 1.0 / (1.0 + jnp.exp(-en))
                        ed_v[i, sl] = sg
                        ed_v[i, sl2] = sg * db_v[i, sl2]
                    return rc

                lax.fori_loop(0, K, row, 0)
                pltpu.make_async_copy(ed_v, acc.at[dst_v], sem_s).start(
                    add=True)
            if write_enew:
                pltpu.sync_copy(ce_v, enew_hbm.at[pl.ds(eoff + base, K)])
            pltpu.make_async_copy(ed0, acc.at[dst0], sem_s).wait()
            pltpu.make_async_copy(ed1, acc.at[dst1], sem_s).wait()

            @pl.when(g + 1 < g1)
            def _():
                issue_idx(g + 1)

            return carry

        lax.fori_loop(g0, g1, chunk, 0)
        plsc.subcore_barrier()
        pltpu.sync_copy(
            acc.at[pl.ds(s * ROWS_PT, ROWS_PT)],
            nd_hbm.at[pl.ds(c * N_ACC + s * ROWS_PT, ROWS_PT)])

    cp = pltpu.CompilerParams()
    if "needs_layout_passes" in pltpu.CompilerParams.__dataclass_fields__:
        cp = dataclasses.replace(cp, needs_layout_passes=False)
    fn = pl.kernel(body, out_type=out_type, mesh=mesh,
                   scratch_types=scratch_types, compiler_params=cp)
    res = fn(ce, src, dst, db, eh)
    return tuple(res) if write_enew else res[0]


def kernel(h, e, edge_index, params):
    src = edge_index[0].astype(jnp.int32)
    dst = edge_index[1].astype(jnp.int32)
    h = _node_emb(h, params["emb_h"])
    layers = params["layers"]
    e_f = e_b = None
    enew = st = prev_bn = None
    y = None
    for l, lp in enumerate(layers):
        last = l == len(layers) - 1
        ah, db, eh = _node_prep(h, lp)
        bias = lp["C"]["b"] + lp["D"]["b"] + lp["E"]["b"]
        if l == 0:
            ce, e_f, e_b = _emb_ce(e, params["emb_e"], lp["C"]["W"], bias)
        elif l == 1:
            res = _ce_fused(e_f, e_b, False, enew, st, prev_bn,
                            lp["C"]["W"], bias, write_e=True)
            ce, e_f, e_b = res
        else:
            res = _ce_fused(e_f, e_b, False, enew, st, prev_bn,
                            lp["C"]["W"], bias, write_e=not last)
            if last:
                ce = res[0]
            else:
                ce, e_f, e_b = res
        ce = ce.reshape(2 * PE, DIM)
        db = db.reshape(2 * N_NODES, DIM)
        eh = eh.reshape(2 * N_NODES, DIM)
        if last:
            nd = _sc_message(ce, src, dst, db, eh, write_enew=False)
            y = _node_final(ah, nd.reshape(2, N_ACC, DIM), h,
                            lp["bn_h"], lp["B"]["b"], params["mlp"])
        else:
            enew, nd = _sc_message(ce, src, dst, db, eh, write_enew=True)
            h = _node_update(ah, nd.reshape(2, N_ACC, DIM), h, lp["bn_h"],
                             lp["B"]["b"])
            enew = enew.reshape(2, PE, DIM)
            st = _e_stats(enew)
            prev_bn = lp["bn_e"]
    return y
